# Initial kernel scaffold; baseline (speedup 1.0000x reference)
import jax
import jax.numpy as jnp
from jax.experimental import pallas as pl


def kernel(x, edge_index, W1, b1, W2, b2, W3, b3):
    raise NotImplementedError("write your pallas kernel here")


# speedup vs baseline: 17.7712x; 17.7712x over previous
import functools

import jax
import jax.numpy as jnp
from jax import lax
from jax.experimental import pallas as pl
from jax.experimental.pallas import tpu as pltpu
from jax.experimental.pallas import tpu_sc as plsc

_NC = 2
_NS = 16
_NW = _NC * _NS
_BC = 16


@functools.lru_cache(maxsize=None)
def _build_deg_kernel(N, C, NBLK):
    RPT = N // _NS
    mesh = plsc.VectorSubcoreMesh(core_axis_name="c", subcore_axis_name="s")

    @functools.partial(
        pl.kernel,
        out_type=jax.ShapeDtypeStruct((2 * N, 128), jnp.float32),
        mesh=mesh,
        scratch_types=[
            pltpu.VMEM((_BC, C), jnp.int32),
            pltpu.VMEM((C, 128), jnp.float32),
            pltpu.VMEM((128, 128), jnp.float32),
            pltpu.VMEM_SHARED((N, 128), jnp.float32),
        ],
    )
    def deg_kernel(dst_hbm, ones_hbm, zeros_hbm, out_hbm,
                   idx_v, ones_v, buf_v, deg_sh):
        c = lax.axis_index("c")
        s = lax.axis_index("s")
        wid = c * _NS + s
        row0 = s * RPT
        pltpu.sync_copy(ones_hbm, ones_v)
        pltpu.sync_copy(zeros_hbm, buf_v)
        for i in range(RPT // 128):
            pltpu.sync_copy(buf_v, deg_sh.at[pl.ds(row0 + i * 128, 128)])
        plsc.subcore_barrier()

        for b in range(NBLK):
            pltpu.sync_copy(dst_hbm.at[wid * NBLK + b], idx_v)
            for k in range(_BC):
                pltpu.sync_copy(ones_v, deg_sh.at[idx_v.at[k]], add=True)
        plsc.subcore_barrier()
        for i in range(RPT // 128):
            pltpu.sync_copy(deg_sh.at[pl.ds(row0 + i * 128, 128)], buf_v)
            pltpu.sync_copy(buf_v,
                            out_hbm.at[pl.ds(c * N + row0 + i * 128, 128)])

    return deg_kernel


@functools.lru_cache(maxsize=None)
def _build_agg_kernel(N, D, C, NBLK):
    RPT = N // _NS
    mesh = plsc.VectorSubcoreMesh(core_axis_name="c", subcore_axis_name="s")

    @functools.partial(
        pl.kernel,
        out_type=jax.ShapeDtypeStruct((2 * N, D), jnp.float32),
        mesh=mesh,
        scratch_types=[
            pltpu.VMEM((_BC, C), jnp.int32),
            pltpu.VMEM((_BC, C), jnp.int32),
            pltpu.VMEM((C, D), jnp.float32),
            pltpu.VMEM((C, D), jnp.float32),
            pltpu.VMEM_SHARED((N, D), jnp.float32),
            pltpu.SemaphoreType.DMA,
            pltpu.SemaphoreType.DMA,
        ],
    )
    def agg_kernel(src_hbm, dst_hbm, y_hbm, zeros_hbm, out_hbm,
                   src_v, dst_v, buf_a, buf_b, acc_sh, sem_a, sem_b):
        c = lax.axis_index("c")
        s = lax.axis_index("s")
        wid = c * _NS + s
        row0 = s * RPT
        pltpu.sync_copy(zeros_hbm, buf_a)
        for i in range(RPT // 128):
            pltpu.sync_copy(buf_a, acc_sh.at[pl.ds(row0 + i * 128, 128)])
        plsc.subcore_barrier()

        for b in range(NBLK):
            pltpu.sync_copy(src_hbm.at[wid * NBLK + b], src_v)
            pltpu.sync_copy(dst_hbm.at[wid * NBLK + b], dst_v)
            for kk in range(_BC // 2):
                k0 = kk * 2
                k1 = k0 + 1
                cp_a = pltpu.async_copy(y_hbm.at[src_v.at[k0]], buf_a, sem_a)
                cp_b = pltpu.async_copy(y_hbm.at[src_v.at[k1]], buf_b, sem_b)
                cp_a.wait()
                pltpu.sync_copy(buf_a, acc_sh.at[dst_v.at[k0]], add=True)
                cp_b.wait()
                pltpu.sync_copy(buf_b, acc_sh.at[dst_v.at[k1]], add=True)
        plsc.subcore_barrier()
        for i in range(RPT // 128):
            pltpu.sync_copy(acc_sh.at[pl.ds(row0 + i * 128, 128)], buf_a)
            pltpu.sync_copy(buf_a,
                            out_hbm.at[pl.ds(c * N + row0 + i * 128, 128)])

    return agg_kernel


_BLK = 400


def _tc0_body(d0, d1, x, w, dis_o, y_o):
    deg = d0[:, 0:1] + d1[:, 0:1] + 1.0
    dis = lax.rsqrt(deg)
    dis_o[...] = dis
    y_o[...] = jnp.dot(x[...], w[...], preferred_element_type=jnp.float32) * dis


def _tc_mid_body(p0, p1, y, dis, b, w, y_o):
    agg = p0[...] + p1[...] + y[...]
    h = jnp.maximum(dis[...] * agg + b[...], 0.0)
    y_o[...] = jnp.dot(h, w[...], preferred_element_type=jnp.float32) * dis[...]


def _tc_fin_body(p0, p1, y, dis, b, o):
    h = dis[...] * (p0[...] + p1[...] + y[...]) + b[...]
    m = jnp.max(h, axis=1, keepdims=True)
    e = jnp.exp(h - m)
    o[...] = (h - m) - jnp.log(jnp.sum(e, axis=1, keepdims=True))


def _row_spec(cols):
    return pl.BlockSpec((_BLK, cols), lambda i: (i, 0))


def _full_spec(shape):
    return pl.BlockSpec(shape, lambda i: (0, 0))


def _tc0(d0, d1, x, w):
    N, D = x.shape
    return pl.pallas_call(
        _tc0_body,
        grid=(N // _BLK,),
        in_specs=[_row_spec(D), _row_spec(D), _row_spec(D), _full_spec((D, D))],
        out_specs=[_row_spec(1), _row_spec(D)],
        out_shape=[jax.ShapeDtypeStruct((N, 1), jnp.float32),
                   jax.ShapeDtypeStruct((N, D), jnp.float32)],
    )(d0, d1, x, w)


def _tc_mid(p0, p1, y, dis, b, w):
    N, D = y.shape
    return pl.pallas_call(
        _tc_mid_body,
        grid=(N // _BLK,),
        in_specs=[_row_spec(D), _row_spec(D), _row_spec(D), _row_spec(1),
                  _full_spec((1, D)), _full_spec((D, D))],
        out_specs=_row_spec(D),
        out_shape=jax.ShapeDtypeStruct((N, D), jnp.float32),
    )(p0, p1, y, dis, b, w)


def _tc_fin(p0, p1, y, dis, b):
    N, D = y.shape
    return pl.pallas_call(
        _tc_fin_body,
        grid=(N // _BLK,),
        in_specs=[_row_spec(D), _row_spec(D), _row_spec(D), _row_spec(1),
                  _full_spec((1, D))],
        out_specs=_row_spec(D),
        out_shape=jax.ShapeDtypeStruct((N, D), jnp.float32),
    )(p0, p1, y, dis, b)


def kernel(x, edge_index, W1, b1, W2, b2, W3, b3):
    N, D = x.shape
    E = edge_index.shape[1]
    C = 128
    NP = ((N + 8 * _NS - 1) // (8 * _NS)) * (8 * _NS)
    CW = C * _BC
    EP = ((E + _NW * CW - 1) // (_NW * CW)) * (_NW * CW)
    NBLK = EP // (_NW * CW)
    npad = EP - E

    pad = jnp.arange(npad, dtype=jnp.int32)
    src_i = jnp.concatenate(
        [edge_index[0].astype(jnp.int32), pad % N]).reshape(_NW * NBLK, _BC, C)
    dst_i = jnp.concatenate(
        [edge_index[1].astype(jnp.int32), N + pad % (NP - N)]
    ).reshape(_NW * NBLK, _BC, C)
    onesC = jnp.ones((C, 128), jnp.float32)
    zerD = jnp.zeros((128, D), jnp.float32)

    deg_k = _build_deg_kernel(NP, C, NBLK)
    agg_k = _build_agg_kernel(NP, D, C, NBLK)

    degp = deg_k(dst_i, onesC, zerD)
    dis, y1 = _tc0(degp[:N], degp[NP:NP + N], x, W1)

    p = agg_k(src_i, dst_i, y1, zerD)
    y2 = _tc_mid(p[:N], p[NP:NP + N], y1, dis, b1.reshape(1, D), W2)

    p = agg_k(src_i, dst_i, y2, zerD)
    y3 = _tc_mid(p[:N], p[NP:NP + N], y2, dis, b2.reshape(1, D), W3)

    p = agg_k(src_i, dst_i, y3, zerD)
    return _tc_fin(p[:N], p[NP:NP + N], y3, dis, b3.reshape(1, D))


# speedup vs baseline: 20.1810x; 1.1356x over previous
import functools

import jax
import jax.numpy as jnp
from jax import lax
from jax.experimental import pallas as pl
from jax.experimental.pallas import tpu as pltpu
from jax.experimental.pallas import tpu_sc as plsc

_NC = 2
_NS = 16
_NW = _NC * _NS
_BC = 16


@functools.lru_cache(maxsize=None)
def _build_deg_kernel(N, C, NBLK):
    RPT = N // _NS
    NZI = RPT // 128
    mesh = plsc.VectorSubcoreMesh(core_axis_name="c", subcore_axis_name="s")

    @functools.partial(
        pl.kernel,
        out_type=jax.ShapeDtypeStruct((2 * N, 128), jnp.float32),
        mesh=mesh,
        scratch_types=[
            pltpu.VMEM((_BC, C), jnp.int32),
            pltpu.VMEM((_BC, C), jnp.int32),
            pltpu.VMEM((_BC, C), jnp.int32),
            pltpu.VMEM((C, 128), jnp.float32),
            pltpu.VMEM((128, 128), jnp.float32),
            pltpu.VMEM_SHARED((N, 128), jnp.float32),
            pltpu.SemaphoreType.DMA,
            pltpu.SemaphoreType.DMA,
            pltpu.SemaphoreType.DMA,
            pltpu.SemaphoreType.DMA,
            pltpu.SemaphoreType.DMA,
        ],
    )
    def deg_kernel(dst_hbm, ones_hbm, zeros_hbm, out_hbm,
                   dst0_v, dst1_v, dst2_v, ones_v, buf_v, deg_sh,
                   ss_a, ss_b, si_0, si_1, si_2):
        c = lax.axis_index("c")
        s = lax.axis_index("s")
        wid = c * _NS + s
        row0 = s * RPT
        dstv = (dst0_v, dst1_v, dst2_v)
        si = (si_0, si_1, si_2)
        ss = (ss_a, ss_b)

        ipend = [None, None, None]
        ipend[0] = pltpu.async_copy(dst_hbm.at[wid * NBLK], dstv[0], si[0])
        if NBLK > 1:
            ipend[1] = pltpu.async_copy(dst_hbm.at[wid * NBLK + 1], dstv[1],
                                        si[1])
        pltpu.sync_copy(ones_hbm, ones_v)
        pltpu.sync_copy(zeros_hbm, buf_v)
        init = []
        for i in range(NZI):
            init.append(pltpu.async_copy(
                buf_v, deg_sh.at[pl.ds(row0 + i * 128, 128)], ss[i % 2]))
        for cp in init:
            cp.wait()
        plsc.subcore_barrier()

        NCHT = NBLK * _BC
        for k in range(NCHT):
            b, pos = divmod(k, _BC)
            if pos == 0:
                ipend[b % 3].wait()
            pltpu.sync_copy(ones_v, deg_sh.at[dstv[b % 3].at[pos]], add=True)
            if pos == 2 and b + 2 < NBLK:
                ipend[(b + 2) % 3] = pltpu.async_copy(
                    dst_hbm.at[wid * NBLK + b + 2], dstv[(b + 2) % 3],
                    si[(b + 2) % 3])
        plsc.subcore_barrier()

        wbufs = (buf_v, ones_v)
        wb = [None] * NZI
        for i in range(NZI):
            if i >= 2:
                wb[i - 2].wait()
            pltpu.sync_copy(deg_sh.at[pl.ds(row0 + i * 128, 128)],
                            wbufs[i % 2])
            wb[i] = pltpu.async_copy(
                wbufs[i % 2],
                out_hbm.at[pl.ds(c * N + row0 + i * 128, 128)], ss[i % 2])
        wb[NZI - 2].wait()
        wb[NZI - 1].wait()

    return deg_kernel


@functools.lru_cache(maxsize=None)
def _build_agg_kernel(N, D, C, NBLK):
    RPT = N // _NS
    NZI = RPT // 128
    NCHT = NBLK * _BC
    mesh = plsc.VectorSubcoreMesh(core_axis_name="c", subcore_axis_name="s")

    @functools.partial(
        pl.kernel,
        out_type=jax.ShapeDtypeStruct((2 * N, D), jnp.float32),
        mesh=mesh,
        scratch_types=[
            pltpu.VMEM((NCHT, C), jnp.int32),
            pltpu.VMEM((_BC, C), jnp.int32),
            pltpu.VMEM((_BC, C), jnp.int32),
            pltpu.VMEM((_BC, C), jnp.int32),
            pltpu.VMEM((C, D), jnp.float32),
            pltpu.VMEM((C, D), jnp.float32),
            pltpu.VMEM_SHARED((N, D), jnp.float32),
            pltpu.SemaphoreType.DMA,
            pltpu.SemaphoreType.DMA,
            pltpu.SemaphoreType.DMA,
            pltpu.SemaphoreType.DMA,
            pltpu.SemaphoreType.DMA,
            pltpu.SemaphoreType.DMA,
            pltpu.SemaphoreType.DMA,
        ],
    )
    def agg_kernel(src_hbm, dst_hbm, y_hbm, zeros_hbm, out_hbm,
                   src_v, dst0_v, dst1_v, dst2_v, buf_a, buf_b, acc_sh,
                   sg_a, sg_b, ss_a, ss_b, si_0, si_1, si_2):
        c = lax.axis_index("c")
        s = lax.axis_index("s")
        wid = c * _NS + s
        row0 = s * RPT
        bufs = (buf_a, buf_b)
        sg = (sg_a, sg_b)
        ss = (ss_a, ss_b)
        dstv = (dst0_v, dst1_v, dst2_v)
        si = (si_0, si_1, si_2)

        ipend = [None, None, None]
        ipend[0] = pltpu.async_copy(dst_hbm.at[wid * NBLK], dstv[0], si[0])
        if NBLK > 1:
            ipend[1] = pltpu.async_copy(dst_hbm.at[wid * NBLK + 1], dstv[1],
                                        si[1])
        pltpu.sync_copy(src_hbm.at[wid], src_v)
        pltpu.sync_copy(zeros_hbm, buf_a)
        init = []
        for i in range(NZI):
            init.append(pltpu.async_copy(
                buf_a, acc_sh.at[pl.ds(row0 + i * 128, 128)], ss[i % 2]))
        for cp in init:
            cp.wait()
        plsc.subcore_barrier()

        g = [None] * (NCHT + 1)
        g[0] = pltpu.async_copy(y_hbm.at[src_v.at[0]], bufs[0], sg[0])
        for k in range(NCHT):
            bk = k % 2
            b, pos = divmod(k, _BC)
            if pos == 0:
                ipend[b % 3].wait()
            g[k].wait()
            if k + 1 < NCHT:
                g[k + 1] = pltpu.async_copy(
                    y_hbm.at[src_v.at[k + 1]], bufs[(k + 1) % 2],
                    sg[(k + 1) % 2])
            pltpu.sync_copy(bufs[bk], acc_sh.at[dstv[b % 3].at[pos]], add=True)
            if pos == 2 and b + 2 < NBLK:
                ipend[(b + 2) % 3] = pltpu.async_copy(
                    dst_hbm.at[wid * NBLK + b + 2], dstv[(b + 2) % 3],
                    si[(b + 2) % 3])
        plsc.subcore_barrier()

        wb = [None] * NZI
        for i in range(NZI):
            if i >= 2:
                wb[i - 2].wait()
            pltpu.sync_copy(acc_sh.at[pl.ds(row0 + i * 128, 128)],
                            bufs[i % 2])
            wb[i] = pltpu.async_copy(
                bufs[i % 2],
                out_hbm.at[pl.ds(c * N + row0 + i * 128, 128)], sg[i % 2])
        wb[NZI - 2].wait()
        wb[NZI - 1].wait()

    return agg_kernel


_BLK = 400


def _tc0_body(d0, d1, x, w, dis_o, y_o):
    deg = d0[:, 0:1] + d1[:, 0:1] + 1.0
    dis = lax.rsqrt(deg)
    dis_o[...] = dis
    y_o[...] = jnp.dot(x[...], w[...], preferred_element_type=jnp.float32) * dis


def _tc_mid_body(p0, p1, y, dis, b, w, y_o):
    agg = p0[...] + p1[...] + y[...]
    h = jnp.maximum(dis[...] * agg + b[...], 0.0)
    y_o[...] = jnp.dot(h, w[...], preferred_element_type=jnp.float32) * dis[...]


def _tc_fin_body(p0, p1, y, dis, b, o):
    h = dis[...] * (p0[...] + p1[...] + y[...]) + b[...]
    m = jnp.max(h, axis=1, keepdims=True)
    e = jnp.exp(h - m)
    o[...] = (h - m) - jnp.log(jnp.sum(e, axis=1, keepdims=True))


def _row_spec(cols):
    return pl.BlockSpec((_BLK, cols), lambda i: (i, 0))


def _full_spec(shape):
    return pl.BlockSpec(shape, lambda i: (0, 0))


def _tc0(d0, d1, x, w):
    N, D = x.shape
    return pl.pallas_call(
        _tc0_body,
        grid=(N // _BLK,),
        in_specs=[_row_spec(D), _row_spec(D), _row_spec(D), _full_spec((D, D))],
        out_specs=[_row_spec(1), _row_spec(D)],
        out_shape=[jax.ShapeDtypeStruct((N, 1), jnp.float32),
                   jax.ShapeDtypeStruct((N, D), jnp.float32)],
    )(d0, d1, x, w)


def _tc_mid(p0, p1, y, dis, b, w):
    N, D = y.shape
    return pl.pallas_call(
        _tc_mid_body,
        grid=(N // _BLK,),
        in_specs=[_row_spec(D), _row_spec(D), _row_spec(D), _row_spec(1),
                  _full_spec((1, D)), _full_spec((D, D))],
        out_specs=_row_spec(D),
        out_shape=jax.ShapeDtypeStruct((N, D), jnp.float32),
    )(p0, p1, y, dis, b, w)


def _tc_fin(p0, p1, y, dis, b):
    N, D = y.shape
    return pl.pallas_call(
        _tc_fin_body,
        grid=(N // _BLK,),
        in_specs=[_row_spec(D), _row_spec(D), _row_spec(D), _row_spec(1),
                  _full_spec((1, D))],
        out_specs=_row_spec(D),
        out_shape=jax.ShapeDtypeStruct((N, D), jnp.float32),
    )(p0, p1, y, dis, b)


def kernel(x, edge_index, W1, b1, W2, b2, W3, b3):
    N, D = x.shape
    E = edge_index.shape[1]
    C = 128
    NP = ((N + 8 * _NS - 1) // (8 * _NS)) * (8 * _NS)
    CW = C * _BC
    EP = ((E + _NW * CW - 1) // (_NW * CW)) * (_NW * CW)
    NBLK = EP // (_NW * CW)
    NCHT = NBLK * _BC
    npad = EP - E

    pad = jnp.arange(npad, dtype=jnp.int32)
    src_i = jnp.concatenate(
        [edge_index[0].astype(jnp.int32), pad % N]).reshape(_NW, NCHT, C)
    dst_i = jnp.concatenate(
        [edge_index[1].astype(jnp.int32), N + pad % (NP - N)]
    ).reshape(_NW * NBLK, _BC, C)
    onesC = jnp.ones((C, 128), jnp.float32)
    zerD = jnp.zeros((128, D), jnp.float32)

    deg_k = _build_deg_kernel(NP, C, NBLK)
    agg_k = _build_agg_kernel(NP, D, C, NBLK)

    degp = deg_k(dst_i, onesC, zerD)
    dis, y1 = _tc0(degp[:N], degp[NP:NP + N], x, W1)

    p = agg_k(src_i, dst_i, y1, zerD)
    y2 = _tc_mid(p[:N], p[NP:NP + N], y1, dis, b1.reshape(1, D), W2)

    p = agg_k(src_i, dst_i, y2, zerD)
    y3 = _tc_mid(p[:N], p[NP:NP + N], y2, dis, b2.reshape(1, D), W3)

    p = agg_k(src_i, dst_i, y3, zerD)
    return _tc_fin(p[:N], p[NP:NP + N], y3, dis, b3.reshape(1, D))
